# baseline (speedup 1.0000x reference)
import functools

import jax
import jax.numpy as jnp
from jax import lax
from jax.experimental import pallas as pl
from jax.experimental.pallas import tpu as pltpu
from jax.experimental.pallas import tpu_sc as plsc

_NC = 2
_NS = 16
_L = 16
_XCHUNK = 4096


def _make_gather_kernel(B, V, D):
    nw = _NC * _NS
    d_per_w = D // nw
    n_chunks = B // _XCHUNK
    split = (V // 2 + 127) // 128 * 128
    h_sizes = (split, V - split)
    mesh = plsc.VectorSubcoreMesh(core_axis_name="c", subcore_axis_name="s")

    @functools.partial(
        pl.kernel,
        out_type=jax.ShapeDtypeStruct((D, B), jnp.float32),
        mesh=mesh,
        scratch_types=[
            pltpu.VMEM((h_sizes[0],), jnp.float32),
            pltpu.VMEM((h_sizes[1],), jnp.float32),
            pltpu.VMEM((_XCHUNK,), jnp.int32),
            pltpu.VMEM((B,), jnp.float32),
            pltpu.SemaphoreType.DMA,
            pltpu.SemaphoreType.DMA,
        ],
        compiler_params=pltpu.CompilerParams(needs_layout_passes=False),
    )
    def gather_kernel(x_hbm, t_hbm, g_hbm, bufA, bufB, x_v, o_v, semA, semB):
        wid = lax.axis_index("s") * _NC + lax.axis_index("c")
        d0 = wid * d_per_w
        bufs = (bufA, bufB)
        sems = (semA, semB)
        lane = lax.iota(jnp.int32, _L)
        split_v = jnp.full((_L,), split, jnp.int32)

        def stage(d, h):
            off = 0 if h == 0 else split
            return pltpu.async_copy(
                t_hbm.at[d, pl.ds(off, h_sizes[h])], bufs[h], sems[h]
            )

        def scan_half(h):
            buf = bufs[h]
            for chunk in range(n_chunks):
                pltpu.sync_copy(x_hbm.at[pl.ds(chunk * _XCHUNK, _XCHUNK)], x_v)
                pos0 = chunk * _XCHUNK

                def body(j, carry):
                    idx = x_v[pl.ds(j * _L, _L)]
                    if h == 0:
                        m = idx < split_v
                        loc = idx
                    else:
                        m = idx >= split_v
                        loc = idx - split_v
                    v = plsc.load_gather(buf, [loc], mask=m)
                    plsc.store_scatter(o_v, [pos0 + j * _L + lane], v, mask=m)
                    return carry

                lax.fori_loop(0, _XCHUNK // _L, body, jnp.int32(0))

        cpA = stage(d0, 0)
        cpB = stage(d0, 1)
        for k in range(d_per_w):
            d = d0 + k
            cpA.wait()
            scan_half(0)
            if k + 1 < d_per_w:
                cpA = stage(d + 1, 0)
            cpB.wait()
            scan_half(1)
            if k + 1 < d_per_w:
                cpB = stage(d + 1, 1)
            pltpu.sync_copy(o_v, g_hbm.at[d])

    return gather_kernel


def _ln_block(g_ref, gamma_ref, beta_ref, o_ref):
    g = g_ref[...]
    mean = jnp.mean(g, axis=0, keepdims=True)
    var = jnp.mean(g * g, axis=0, keepdims=True) - mean * mean
    rstd = lax.rsqrt(var + jnp.float32(1e-5))
    o_ref[...] = (g - mean) * rstd * gamma_ref[...] + beta_ref[...]


def _make_ln_kernel(B, D, blk=2048):
    grid = (B // blk,)
    return pl.pallas_call(
        _ln_block,
        grid=grid,
        in_specs=[
            pl.BlockSpec((D, blk), lambda i: (0, i)),
            pl.BlockSpec((D, 1), lambda i: (0, 0)),
            pl.BlockSpec((D, 1), lambda i: (0, 0)),
        ],
        out_specs=pl.BlockSpec((D, blk), lambda i: (0, i)),
        out_shape=jax.ShapeDtypeStruct((D, B), jnp.float32),
    )


def kernel(x, table, gamma, beta):
    B = x.shape[0]
    V, D = table.shape
    gathered = _make_gather_kernel(B, V, D)(x.astype(jnp.int32), table.T)
    out_t = _make_ln_kernel(B, D)(
        gathered, gamma.reshape(D, 1), beta.reshape(D, 1)
    )
    return out_t.T


# speedup vs baseline: 2.1488x; 2.1488x over previous
import functools

import jax
import jax.numpy as jnp
from jax import lax
from jax.experimental import pallas as pl
from jax.experimental.pallas import tpu as pltpu
from jax.experimental.pallas import tpu_sc as plsc

_NC = 2
_NS = 16
_L = 16
_XCHUNK = 4096


def _make_gather_kernel(B, V, D):
    nw = _NC * _NS
    d_per_w = D // nw
    n_chunks = B // _XCHUNK
    mesh = plsc.VectorSubcoreMesh(core_axis_name="c", subcore_axis_name="s")

    @functools.partial(
        pl.kernel,
        out_type=jax.ShapeDtypeStruct((D, B), jnp.float32),
        mesh=mesh,
        scratch_types=[
            pltpu.VMEM((V,), jnp.float32),
            pltpu.VMEM((B,), jnp.int32),
            pltpu.VMEM((_XCHUNK,), jnp.float32),
            pltpu.VMEM((_XCHUNK,), jnp.float32),
            pltpu.SemaphoreType.DMA,
            pltpu.SemaphoreType.DMA,
        ],
        compiler_params=pltpu.CompilerParams(needs_layout_passes=False),
    )
    def gather_kernel(x_hbm, t_hbm, g_hbm, row_v, x_v, o0, o1, sem0, sem1):
        wid = lax.axis_index("s") * _NC + lax.axis_index("c")
        d0 = wid * d_per_w
        obufs = (o0, o1)
        sems = (sem0, sem1)

        pltpu.sync_copy(x_hbm, x_v)

        pending = []

        def scan_chunk(buf, chunk):
            base = chunk * _XCHUNK

            @plsc.parallel_loop(0, _XCHUNK // _L, unroll=8)
            def body(j):
                idx = x_v[pl.ds(base + j * _L, _L)]
                buf[pl.ds(j * _L, _L)] = plsc.load_gather(row_v, [idx])

        for k in range(d_per_w):
            d = d0 + k
            pltpu.sync_copy(t_hbm.at[d], row_v)
            for chunk in range(n_chunks):
                slot = (k * n_chunks + chunk) % 2
                if len(pending) >= 2:
                    pending.pop(0).wait()
                scan_chunk(obufs[slot], chunk)
                cp = pltpu.async_copy(
                    obufs[slot], g_hbm.at[d, pl.ds(chunk * _XCHUNK, _XCHUNK)],
                    sems[slot],
                )
                pending.append(cp)
        for cp in pending:
            cp.wait()

    return gather_kernel


def _ln_block(g_ref, gamma_ref, beta_ref, o_ref):
    g = g_ref[...]
    mean = jnp.mean(g, axis=0, keepdims=True)
    var = jnp.mean(g * g, axis=0, keepdims=True) - mean * mean
    rstd = lax.rsqrt(var + jnp.float32(1e-5))
    o_ref[...] = (g - mean) * rstd * gamma_ref[...] + beta_ref[...]


def _make_ln_kernel(B, D, blk=2048):
    grid = (B // blk,)
    return pl.pallas_call(
        _ln_block,
        grid=grid,
        in_specs=[
            pl.BlockSpec((D, blk), lambda i: (0, i)),
            pl.BlockSpec((D, 1), lambda i: (0, 0)),
            pl.BlockSpec((D, 1), lambda i: (0, 0)),
        ],
        out_specs=pl.BlockSpec((D, blk), lambda i: (0, i)),
        out_shape=jax.ShapeDtypeStruct((D, B), jnp.float32),
    )


def kernel(x, table, gamma, beta):
    B = x.shape[0]
    V, D = table.shape
    gathered = _make_gather_kernel(B, V, D)(x.astype(jnp.int32), table.T)
    out_t = _make_ln_kernel(B, D)(
        gathered, gamma.reshape(D, 1), beta.reshape(D, 1)
    )
    return out_t.T


# speedup vs baseline: 2.2322x; 1.0388x over previous
import functools

import jax
import jax.numpy as jnp
from jax import lax
from jax.experimental import pallas as pl
from jax.experimental.pallas import tpu as pltpu
from jax.experimental.pallas import tpu_sc as plsc

_NC = 2
_NS = 16
_L = 16
_XCHUNK = 4096


def _make_gather_kernel(B, V, D):
    nw = _NC * _NS
    d_per_w = D // nw
    n_chunks = B // _XCHUNK
    mesh = plsc.VectorSubcoreMesh(core_axis_name="c", subcore_axis_name="s")

    @functools.partial(
        pl.kernel,
        out_type=jax.ShapeDtypeStruct((D, B), jnp.float32),
        mesh=mesh,
        scratch_types=[
            pltpu.VMEM((V,), jnp.float32),
            pltpu.VMEM((B,), jnp.int32),
            pltpu.VMEM((_XCHUNK,), jnp.float32),
            pltpu.VMEM((_XCHUNK,), jnp.float32),
            pltpu.SemaphoreType.DMA,
            pltpu.SemaphoreType.DMA,
        ],
        compiler_params=pltpu.CompilerParams(needs_layout_passes=False),
    )
    def gather_kernel(x_hbm, t_hbm, g_hbm, row_v, x_v, o0, o1, sem0, sem1):
        wid = lax.axis_index("s") * _NC + lax.axis_index("c")
        d0 = wid * d_per_w
        obufs = (o0, o1)
        sems = (sem0, sem1)

        pltpu.sync_copy(x_hbm, x_v)

        pending = []

        def scan_chunk(buf, chunk):
            base = chunk * _XCHUNK

            @plsc.parallel_loop(0, _XCHUNK // _L, unroll=8)
            def body(j):
                idx = x_v[pl.ds(base + j * _L, _L)]
                buf[pl.ds(j * _L, _L)] = plsc.load_gather(row_v, [idx])

        for k in range(d_per_w):
            d = d0 + k
            pltpu.sync_copy(t_hbm.at[d], row_v)
            for chunk in range(n_chunks):
                slot = (k * n_chunks + chunk) % 2
                if len(pending) >= 2:
                    pending.pop(0).wait()
                scan_chunk(obufs[slot], chunk)
                cp = pltpu.async_copy(
                    obufs[slot], g_hbm.at[d, pl.ds(chunk * _XCHUNK, _XCHUNK)],
                    sems[slot],
                )
                pending.append(cp)
        for cp in pending:
            cp.wait()

    return gather_kernel


def _ln_block(g_ref, gamma_ref, beta_ref, o_ref):
    g = g_ref[...]
    mean = jnp.mean(g, axis=0, keepdims=True)
    var = jnp.mean(g * g, axis=0, keepdims=True) - mean * mean
    rstd = lax.rsqrt(var + jnp.float32(1e-5))
    o_ref[...] = (g - mean) * rstd * gamma_ref[...] + beta_ref[...]


def _make_ln_kernel(B, D, blk=4096):
    grid = (B // blk,)
    return pl.pallas_call(
        _ln_block,
        grid=grid,
        in_specs=[
            pl.BlockSpec((D, blk), lambda i: (0, i)),
            pl.BlockSpec((D, 1), lambda i: (0, 0)),
            pl.BlockSpec((D, 1), lambda i: (0, 0)),
        ],
        out_specs=pl.BlockSpec((D, blk), lambda i: (0, i)),
        out_shape=jax.ShapeDtypeStruct((D, B), jnp.float32),
    )


def kernel(x, table, gamma, beta):
    B = x.shape[0]
    V, D = table.shape
    gathered = _make_gather_kernel(B, V, D)(x.astype(jnp.int32), table.T)
    out_t = _make_ln_kernel(B, D)(
        gathered, gamma.reshape(D, 1), beta.reshape(D, 1)
    )
    return out_t.T


# speedup vs baseline: 2.2608x; 1.0128x over previous
import functools

import jax
import jax.numpy as jnp
from jax import lax
from jax.experimental import pallas as pl
from jax.experimental.pallas import tpu as pltpu
from jax.experimental.pallas import tpu_sc as plsc

_NC = 2
_NS = 16
_L = 16
_XCHUNK = 4096


def _make_gather_kernel(B, V, D):
    nw = _NC * _NS
    d_per_w = D // nw
    n_chunks = B // _XCHUNK
    mesh = plsc.VectorSubcoreMesh(core_axis_name="c", subcore_axis_name="s")

    @functools.partial(
        pl.kernel,
        out_type=jax.ShapeDtypeStruct((D, B), jnp.float32),
        mesh=mesh,
        scratch_types=[
            pltpu.VMEM((V,), jnp.float32),
            pltpu.VMEM((B,), jnp.int32),
            pltpu.VMEM((_XCHUNK,), jnp.float32),
            pltpu.VMEM((_XCHUNK,), jnp.float32),
            pltpu.SemaphoreType.DMA,
            pltpu.SemaphoreType.DMA,
            pltpu.SemaphoreType.DMA,
        ],
        compiler_params=pltpu.CompilerParams(needs_layout_passes=False),
    )
    def gather_kernel(x_hbm, t_hbm, g_hbm, row_v, x_v, o0, o1, sem0, sem1,
                      sem_row):
        wid = lax.axis_index("s") * _NC + lax.axis_index("c")
        d0 = wid * d_per_w
        obufs = (o0, o1)
        sems = (sem0, sem1)

        def stage_row(d):
            pltpu.async_copy(t_hbm.at[d], row_v, sem_row).wait()

        pltpu.sync_copy(x_hbm, x_v)

        pending = []

        def scan_chunk(buf, chunk):
            base = chunk * _XCHUNK

            @plsc.parallel_loop(0, _XCHUNK // _L, unroll=8)
            def body(j):
                idx = x_v[pl.ds(base + j * _L, _L)]
                buf[pl.ds(j * _L, _L)] = plsc.load_gather(row_v, [idx])

        for k in range(d_per_w):
            d = d0 + k
            stage_row(d)
            for chunk in range(n_chunks):
                slot = (k * n_chunks + chunk) % 2
                if len(pending) >= 2:
                    pending.pop(0).wait()
                scan_chunk(obufs[slot], chunk)
                cp = pltpu.async_copy(
                    obufs[slot], g_hbm.at[d, pl.ds(chunk * _XCHUNK, _XCHUNK)],
                    sems[slot],
                )
                pending.append(cp)
        for cp in pending:
            cp.wait()

    return gather_kernel


def _ln_block(g_ref, gamma_ref, beta_ref, o_ref):
    g = g_ref[...]
    mean = jnp.mean(g, axis=0, keepdims=True)
    var = jnp.mean(g * g, axis=0, keepdims=True) - mean * mean
    rstd = lax.rsqrt(var + jnp.float32(1e-5))
    o_ref[...] = (g - mean) * rstd * gamma_ref[...] + beta_ref[...]


def _make_ln_kernel(B, D, blk=8192):
    grid = (B // blk,)
    return pl.pallas_call(
        _ln_block,
        grid=grid,
        in_specs=[
            pl.BlockSpec((D, blk), lambda i: (0, i)),
            pl.BlockSpec((D, 1), lambda i: (0, 0)),
            pl.BlockSpec((D, 1), lambda i: (0, 0)),
        ],
        out_specs=pl.BlockSpec((D, blk), lambda i: (0, i)),
        out_shape=jax.ShapeDtypeStruct((D, B), jnp.float32),
    )


def kernel(x, table, gamma, beta):
    B = x.shape[0]
    V, D = table.shape
    gathered = _make_gather_kernel(B, V, D)(x.astype(jnp.int32), table.T)
    out_t = _make_ln_kernel(B, D)(
        gathered, gamma.reshape(D, 1), beta.reshape(D, 1)
    )
    return out_t.T


# speedup vs baseline: 2.2625x; 1.0008x over previous
import functools

import jax
import jax.numpy as jnp
from jax import lax
from jax.experimental import pallas as pl
from jax.experimental.pallas import tpu as pltpu
from jax.experimental.pallas import tpu_sc as plsc

_NC = 2
_NS = 16
_L = 16
_XCHUNK = 4096


def _make_gather_kernel(B, V, D):
    nw = _NC * _NS
    d_per_w = D // nw
    n_chunks = B // _XCHUNK
    mesh = plsc.VectorSubcoreMesh(core_axis_name="c", subcore_axis_name="s")

    @functools.partial(
        pl.kernel,
        out_type=jax.ShapeDtypeStruct((D, B), jnp.float32),
        mesh=mesh,
        scratch_types=[
            pltpu.VMEM((V,), jnp.float32),
            pltpu.VMEM((B,), jnp.int32),
            pltpu.VMEM((_XCHUNK,), jnp.float32),
            pltpu.VMEM((_XCHUNK,), jnp.float32),
            pltpu.SemaphoreType.DMA,
            pltpu.SemaphoreType.DMA,
            pltpu.SemaphoreType.DMA,
        ],
        compiler_params=pltpu.CompilerParams(needs_layout_passes=False),
    )
    def gather_kernel(x_hbm, t_hbm, g_hbm, row_v, x_v, o0, o1, sem0, sem1,
                      sem_row):
        wid = lax.axis_index("s") * _NC + lax.axis_index("c")
        d0 = wid * d_per_w
        obufs = (o0, o1)
        sems = (sem0, sem1)

        cp_row = pltpu.async_copy(t_hbm.at[d0], row_v, sem_row)
        pltpu.sync_copy(x_hbm, x_v)

        pending = []

        def scan_chunk(buf, chunk):
            base = chunk * _XCHUNK

            @plsc.parallel_loop(0, _XCHUNK // _L, unroll=16)
            def body(j):
                idx = x_v[pl.ds(base + j * _L, _L)]
                buf[pl.ds(j * _L, _L)] = plsc.load_gather(row_v, [idx])

        for k in range(d_per_w):
            d = d0 + k
            if k > 0:
                cp_row = pltpu.async_copy(t_hbm.at[d], row_v, sem_row)
            cp_row.wait()
            for chunk in range(n_chunks):
                slot = (k * n_chunks + chunk) % 2
                if len(pending) >= 2:
                    pending.pop(0).wait()
                scan_chunk(obufs[slot], chunk)
                cp = pltpu.async_copy(
                    obufs[slot], g_hbm.at[d, pl.ds(chunk * _XCHUNK, _XCHUNK)],
                    sems[slot],
                )
                pending.append(cp)
        for cp in pending:
            cp.wait()

    return gather_kernel


def _ln_block(g_ref, gamma_ref, beta_ref, o_ref):
    g = g_ref[...]
    mean = jnp.mean(g, axis=0, keepdims=True)
    var = jnp.mean(g * g, axis=0, keepdims=True) - mean * mean
    rstd = lax.rsqrt(var + jnp.float32(1e-5))
    o_ref[...] = (g - mean) * rstd * gamma_ref[...] + beta_ref[...]


def _make_ln_kernel(B, D, blk=8192):
    grid = (B // blk,)
    return pl.pallas_call(
        _ln_block,
        grid=grid,
        in_specs=[
            pl.BlockSpec((D, blk), lambda i: (0, i)),
            pl.BlockSpec((D, 1), lambda i: (0, 0)),
            pl.BlockSpec((D, 1), lambda i: (0, 0)),
        ],
        out_specs=pl.BlockSpec((D, blk), lambda i: (0, i)),
        out_shape=jax.ShapeDtypeStruct((D, B), jnp.float32),
    )


def kernel(x, table, gamma, beta):
    B = x.shape[0]
    V, D = table.shape
    gathered = _make_gather_kernel(B, V, D)(x.astype(jnp.int32), table.T)
    out_t = _make_ln_kernel(B, D)(
        gathered, gamma.reshape(D, 1), beta.reshape(D, 1)
    )
    return out_t.T


# speedup vs baseline: 2.2628x; 1.0001x over previous
import functools

import jax
import jax.numpy as jnp
from jax import lax
from jax.experimental import pallas as pl
from jax.experimental.pallas import tpu as pltpu
from jax.experimental.pallas import tpu_sc as plsc

_NC = 2
_NS = 16
_L = 16
_XCHUNK = 4096


def _make_gather_kernel(B, V, D):
    nw = _NC * _NS
    d_per_w = D // nw
    n_chunks = B // _XCHUNK
    mesh = plsc.VectorSubcoreMesh(core_axis_name="c", subcore_axis_name="s")

    @functools.partial(
        pl.kernel,
        out_type=jax.ShapeDtypeStruct((D, B), jnp.float32),
        mesh=mesh,
        scratch_types=[
            pltpu.VMEM((V,), jnp.float32),
            pltpu.VMEM((B,), jnp.int32),
            pltpu.VMEM((_XCHUNK,), jnp.float32),
            pltpu.VMEM((_XCHUNK,), jnp.float32),
            pltpu.SemaphoreType.DMA,
            pltpu.SemaphoreType.DMA,
            pltpu.SemaphoreType.DMA,
        ],
        compiler_params=pltpu.CompilerParams(
            needs_layout_passes=False, disable_bounds_checks=True
        ),
    )
    def gather_kernel(x_hbm, t_hbm, g_hbm, row_v, x_v, o0, o1, sem0, sem1,
                      sem_row):
        wid = lax.axis_index("s") * _NC + lax.axis_index("c")
        d0 = wid * d_per_w
        obufs = (o0, o1)
        sems = (sem0, sem1)

        cp_row = pltpu.async_copy(t_hbm.at[d0], row_v, sem_row)
        pltpu.sync_copy(x_hbm, x_v)

        pending = []

        def scan_chunk(buf, chunk):
            base = chunk * _XCHUNK

            @plsc.parallel_loop(0, _XCHUNK // _L, unroll=16)
            def body(j):
                idx = x_v[pl.ds(base + j * _L, _L)]
                buf[pl.ds(j * _L, _L)] = plsc.load_gather(row_v, [idx])

        for k in range(d_per_w):
            d = d0 + k
            if k > 0:
                cp_row = pltpu.async_copy(t_hbm.at[d], row_v, sem_row)
            cp_row.wait()
            for chunk in range(n_chunks):
                slot = (k * n_chunks + chunk) % 2
                if len(pending) >= 2:
                    pending.pop(0).wait()
                scan_chunk(obufs[slot], chunk)
                cp = pltpu.async_copy(
                    obufs[slot], g_hbm.at[d, pl.ds(chunk * _XCHUNK, _XCHUNK)],
                    sems[slot],
                )
                pending.append(cp)
        for cp in pending:
            cp.wait()

    return gather_kernel


def _ln_block(g_ref, gamma_ref, beta_ref, o_ref):
    g = g_ref[...]
    mean = jnp.mean(g, axis=0, keepdims=True)
    var = jnp.mean(g * g, axis=0, keepdims=True) - mean * mean
    rstd = lax.rsqrt(var + jnp.float32(1e-5))
    o_ref[...] = (g - mean) * rstd * gamma_ref[...] + beta_ref[...]


def _make_ln_kernel(B, D, blk=8192):
    grid = (B // blk,)
    return pl.pallas_call(
        _ln_block,
        grid=grid,
        in_specs=[
            pl.BlockSpec((D, blk), lambda i: (0, i)),
            pl.BlockSpec((D, 1), lambda i: (0, 0)),
            pl.BlockSpec((D, 1), lambda i: (0, 0)),
        ],
        out_specs=pl.BlockSpec((D, blk), lambda i: (0, i)),
        out_shape=jax.ShapeDtypeStruct((D, B), jnp.float32),
    )


def kernel(x, table, gamma, beta):
    B = x.shape[0]
    V, D = table.shape
    gathered = _make_gather_kernel(B, V, D)(x.astype(jnp.int32), table.T)
    out_t = _make_ln_kernel(B, D)(
        gathered, gamma.reshape(D, 1), beta.reshape(D, 1)
    )
    return out_t.T
